# baseline (speedup 1.0000x reference)
import jax
import jax.numpy as jnp
from jax import lax
from jax.experimental import pallas as pl
from jax.experimental.pallas import tpu as pltpu
from jax.experimental.pallas import tpu_sc as plsc

N = 10000
E = 320000
D = 128
C = 16

NTILES = 16
LANES = 16

ROWS_PER_TILE = 640
NACC = NTILES * ROWS_PER_TILE
TRASH = N

CHUNK = 128
NCH = 158
NCHG = NCH + 2
EPT = NCH * CHUNK
E_PAD = NTILES * EPT

W = 80


def _vgather16(v, idx):
    return lax.gather(
        v, idx[:, None],
        lax.GatherDimensionNumbers(
            offset_dims=(), collapsed_slice_dims=(0,), start_index_map=(0,)),
        (1,), mode=lax.GatherScatterMode.PROMISE_IN_BOUNDS)


def _hop_body(t0, t1, src3d, dst3d, out0, out1,
              src_idx, dst_idx, rows0, rows1, rows2, ones_b, cz, cntb,
              acc, cnt_sh,
              sem_g0, sem_g1, sem_g2, sem_s0, sem_s1, sem_s2):
    cid = lax.axis_index("c")
    sid = lax.axis_index("s")
    base_row = sid * ROWS_PER_TILE

    zero16 = jnp.zeros((LANES,), jnp.float32)
    one16 = jnp.ones((LANES,), jnp.float32)
    for r in range(CHUNK):
        for cc in range(W // LANES):
            rows0[r, pl.ds(cc * LANES, LANES)] = zero16
    for i in range(CHUNK // LANES):
        ones_b[pl.ds(i * LANES, LANES)] = one16
    for i in range(ROWS_PER_TILE // LANES):
        cz[pl.ds(i * LANES, LANES)] = zero16

    pltpu.sync_copy(cz, cnt_sh.at[pl.ds(base_row, ROWS_PER_TILE)])
    for k in range(ROWS_PER_TILE // CHUNK):
        pltpu.sync_copy(rows0, acc.at[pl.ds(base_row + k * CHUNK, CHUNK)])

    pltpu.sync_copy(src3d.at[sid], src_idx)
    pltpu.sync_copy(dst3d.at[sid], dst_idx)

    bufs = (rows0, rows1, rows2)
    gsems = (sem_g0, sem_g1, sem_g2)
    ssems = (sem_s0, sem_s1, sem_s2)

    def gather(j, b):
        @pl.when(cid == 0)
        def _():
            pltpu.async_copy(t0.at[src_idx.at[j, 0]],
                             bufs[b].at[pl.ds(0, 64)], gsems[b])
            pltpu.async_copy(t0.at[src_idx.at[j, 1]],
                             bufs[b].at[pl.ds(64, 64)], gsems[b])

        @pl.when(cid == 1)
        def _():
            pltpu.async_copy(t1.at[src_idx.at[j, 0]],
                             bufs[b].at[pl.ds(0, 64)], gsems[b])
            pltpu.async_copy(t1.at[src_idx.at[j, 1]],
                             bufs[b].at[pl.ds(64, 64)], gsems[b])

    def gather_wait(b):
        pltpu.make_async_copy(
            t0.at[src_idx.at[0, 0]], bufs[b].at[pl.ds(0, 64)], gsems[b]).wait()
        pltpu.make_async_copy(
            t0.at[src_idx.at[0, 0]], bufs[b].at[pl.ds(0, 64)], gsems[b]).wait()

    def scatter(j, b):
        dij = dst_idx.at[j]
        pltpu.async_copy(bufs[b], acc.at[dij], ssems[b], add=True)
        pltpu.async_copy(ones_b, cnt_sh.at[dij], ssems[b], add=True)

    def scatter_wait(b):
        pltpu.make_async_copy(bufs[b], acc.at[dst_idx.at[0]], ssems[b]).wait()
        pltpu.make_async_copy(ones_b, cnt_sh.at[dst_idx.at[0]], ssems[b]).wait()

    gather(0, 0)
    gather(1, 1)

    plsc.subcore_barrier()

    gather_wait(0)
    scatter(0, 0)
    gather(2, 2)
    gather_wait(1)
    scatter(1, 1)
    scatter_wait(0)
    gather(3, 0)

    def acc_step(i, carry):
        for b in range(3):
            j = 3 * i + 2 + b
            bj = (2 + b) % 3
            gather_wait(bj)
            scatter(j, bj)
            scatter_wait((bj + 2) % 3)
            gather(j + 2, (bj + 2) % 3)
        return carry

    lax.fori_loop(0, (NCH - 2) // 3, acc_step, 0)

    scatter_wait((NCH - 1) % 3)
    gather_wait(NCH % 3)
    gather_wait((NCH + 1) % 3)

    plsc.subcore_barrier()

    def norm_step(k, carry):
        chunk_base = base_row + k * CHUNK
        pltpu.sync_copy(cnt_sh.at[pl.ds(chunk_base, CHUNK)], cntb)
        pltpu.sync_copy(acc.at[pl.ds(chunk_base, CHUNK)], rows0)
        for g in range(CHUNK // LANES):
            c16 = cntb[pl.ds(g * LANES, LANES)]
            rec16 = 1.0 / jnp.maximum(c16, 1.0)
            for p in range(LANES):
                r = g * LANES + p
                rec = _vgather16(rec16, jnp.full((LANES,), p, jnp.int32))
                for cc in range(W // LANES):
                    sl = pl.ds(cc * LANES, LANES)
                    rows0[r, sl] = rows0[r, sl] * rec

        @pl.when(cid == 0)
        def _():
            pltpu.sync_copy(rows0, out0.at[pl.ds(chunk_base, CHUNK)])

        @pl.when(cid == 1)
        def _():
            pltpu.sync_copy(rows0, out1.at[pl.ds(chunk_base, CHUNK)])

        return carry

    lax.fori_loop(0, ROWS_PER_TILE // CHUNK, norm_step, 0)


_hop = pl.kernel(
    _hop_body,
    out_type=(
        jax.ShapeDtypeStruct((NACC, W), jnp.float32),
        jax.ShapeDtypeStruct((NACC, W), jnp.float32),
    ),
    mesh=plsc.VectorSubcoreMesh(core_axis_name="c", subcore_axis_name="s"),
    scratch_types=(
        pltpu.VMEM((NCHG, 2, CHUNK // 2), jnp.int32),
        pltpu.VMEM((NCHG, CHUNK), jnp.int32),
        pltpu.VMEM((CHUNK, W), jnp.float32),
        pltpu.VMEM((CHUNK, W), jnp.float32),
        pltpu.VMEM((CHUNK, W), jnp.float32),
        pltpu.VMEM((CHUNK,), jnp.float32),
        pltpu.VMEM((ROWS_PER_TILE,), jnp.float32),
        pltpu.VMEM((CHUNK,), jnp.float32),
        pltpu.VMEM_SHARED((NACC, W), jnp.float32),
        pltpu.VMEM_SHARED((NACC,), jnp.float32),
        pltpu.SemaphoreType.DMA,
        pltpu.SemaphoreType.DMA,
        pltpu.SemaphoreType.DMA,
        pltpu.SemaphoreType.DMA,
        pltpu.SemaphoreType.DMA,
        pltpu.SemaphoreType.DMA,
    ),
    compiler_params=pltpu.CompilerParams(use_tc_tiling_on_sc=False),
)


def _pad_edges(src, dst):
    npad = E_PAD - E
    pad_src = (jnp.arange(npad, dtype=jnp.int32) % N)
    pad_dst = TRASH + (jnp.arange(npad, dtype=jnp.int32) % (NACC - N))
    s = jnp.concatenate([src, pad_src]).reshape(NTILES, NCH, 2, CHUNK // 2)
    d = jnp.concatenate([dst, pad_dst]).reshape(NTILES, NCH, CHUNK)
    return (jnp.concatenate([s, jnp.zeros((NTILES, 2, 2, CHUNK // 2),
                                           jnp.int32)], axis=1),
            jnp.concatenate([d, jnp.zeros((NTILES, 2, CHUNK),
                                          jnp.int32)], axis=1))


def kernel(x, edge_index_pa, edge_index_ap, y):
    rowpad = ((0, NACC - N), (0, 0))
    t0 = jnp.pad(x[:, : W - C], ((0, NACC - N), (0, C)))
    t1 = jnp.pad(jnp.concatenate([x[:, W - C:], y], axis=1), rowpad)

    pa_s, pa_d = _pad_edges(edge_index_pa[0], edge_index_pa[1])
    ap_s, ap_d = _pad_edges(edge_index_ap[0], edge_index_ap[1])

    h0, h1 = _hop(t0, t1, pa_s, pa_d)
    m0, m1 = _hop(h0, h1, ap_s, ap_d)

    nf = W - C
    mp = jnp.concatenate([m0[:N, :nf], m1[:N, :nf]], axis=1)
    lp = m1[:N, nf:]
    return (mp, lp)


# speedup vs baseline: 1.0308x; 1.0308x over previous
import jax
import jax.numpy as jnp
from jax import lax
from jax.experimental import pallas as pl
from jax.experimental.pallas import tpu as pltpu
from jax.experimental.pallas import tpu_sc as plsc

N = 10000
E = 320000
D = 128
C = 16

NTILES = 16
LANES = 16

ROWS_PER_TILE = 640
NACC = NTILES * ROWS_PER_TILE
TRASH = N

CHUNK = 128
NCH = 160
BLK = 10
NBLK = NCH // BLK
EPT = NCH * CHUNK
E_PAD = NTILES * EPT

W = 80
NORM_CH = ROWS_PER_TILE // CHUNK


def _vgather16(v, idx):
    return lax.gather(
        v, idx[:, None],
        lax.GatherDimensionNumbers(
            offset_dims=(), collapsed_slice_dims=(0,), start_index_map=(0,)),
        (1,), mode=lax.GatherScatterMode.PROMISE_IN_BOUNDS)


def _hop_body(t0, t1, src4d, dst4d, out0, out1,
              sidx0, sidx1, didx0, didx1, rows0, rows1, ones_b, cz, cntb,
              tbl, acc, cnt_sh,
              sem_g0, sem_g1, sem_s0, sem_s1, sem_i0, sem_i1):
    cid = lax.axis_index("c")
    sid = lax.axis_index("s")
    base_row = sid * ROWS_PER_TILE

    sidx = (sidx0, sidx1)
    didx = (didx0, didx1)
    bufs = (rows0, rows1)
    gsems = (sem_g0, sem_g1)
    ssems = (sem_s0, sem_s1)
    isems = (sem_i0, sem_i1)

    zero16 = jnp.zeros((LANES,), jnp.float32)
    one16 = jnp.ones((LANES,), jnp.float32)
    for r in range(CHUNK):
        for cc in range(W // LANES):
            rows0[r, pl.ds(cc * LANES, LANES)] = zero16
    for i in range(CHUNK // LANES):
        ones_b[pl.ds(i * LANES, LANES)] = one16
    for i in range(ROWS_PER_TILE // LANES):
        cz[pl.ds(i * LANES, LANES)] = zero16

    rslice = pl.ds(base_row, ROWS_PER_TILE)

    @pl.when(cid == 0)
    def _():
        pltpu.sync_copy(t0.at[rslice], tbl.at[rslice])

    @pl.when(cid == 1)
    def _():
        pltpu.sync_copy(t1.at[rslice], tbl.at[rslice])

    pltpu.sync_copy(cz, cnt_sh.at[rslice])
    for k in range(NORM_CH):
        pltpu.sync_copy(rows0, acc.at[pl.ds(base_row + k * CHUNK, CHUNK)])

    pltpu.sync_copy(src4d.at[sid, 0], sidx0)
    pltpu.sync_copy(dst4d.at[sid, 0], didx0)

    def gather(j):
        ib = (j // BLK) % 2
        pltpu.async_copy(tbl.at[sidx[ib].at[j % BLK]], bufs[j % 2],
                         gsems[j % 2])

    def gather_wait(j):
        pltpu.make_async_copy(
            tbl.at[sidx0.at[0]], bufs[j % 2], gsems[j % 2]).wait()

    def scatter(j):
        ib, b = (j // BLK) % 2, j % 2
        dij = didx[ib].at[j % BLK]
        pltpu.async_copy(bufs[b], acc.at[dij], ssems[b], add=True)
        pltpu.async_copy(ones_b, cnt_sh.at[dij], ssems[b], add=True)

    def scatter_wait(j):
        b = j % 2
        pltpu.make_async_copy(bufs[b], acc.at[didx0.at[0]], ssems[b]).wait()
        pltpu.make_async_copy(ones_b, cnt_sh.at[didx0.at[0]], ssems[b]).wait()

    plsc.subcore_barrier()

    gather(0)
    gather(1)

    for j in range(NCH):
        B = j // BLK
        if j % BLK == 0 and B + 1 < NBLK:
            nb = (B + 1) % 2
            pltpu.async_copy(src4d.at[sid, B + 1], sidx[nb], isems[nb])
            pltpu.async_copy(dst4d.at[sid, B + 1], didx[nb], isems[nb])
        if j % BLK == BLK - 2 and B + 1 < NBLK:
            nb = (B + 1) % 2
            pltpu.make_async_copy(src4d.at[sid, 0], sidx[nb], isems[nb]).wait()
            pltpu.make_async_copy(dst4d.at[sid, 0], didx[nb], isems[nb]).wait()
        gather_wait(j)
        scatter(j)
        scatter_wait(j)
        if j + 2 < NCH:
            gather(j + 2)

    plsc.subcore_barrier()

    def norm_step(k, carry):
        chunk_base = base_row + k * CHUNK
        pltpu.sync_copy(cnt_sh.at[pl.ds(chunk_base, CHUNK)], cntb)
        pltpu.sync_copy(acc.at[pl.ds(chunk_base, CHUNK)], rows0)
        for g in range(CHUNK // LANES):
            c16 = cntb[pl.ds(g * LANES, LANES)]
            rec16 = 1.0 / jnp.maximum(c16, 1.0)
            for p in range(LANES):
                r = g * LANES + p
                rec = _vgather16(rec16, jnp.full((LANES,), p, jnp.int32))
                for cc in range(W // LANES):
                    sl = pl.ds(cc * LANES, LANES)
                    rows0[r, sl] = rows0[r, sl] * rec

        @pl.when(cid == 0)
        def _():
            pltpu.sync_copy(rows0, out0.at[pl.ds(chunk_base, CHUNK)])

        @pl.when(cid == 1)
        def _():
            pltpu.sync_copy(rows0, out1.at[pl.ds(chunk_base, CHUNK)])

        return carry

    lax.fori_loop(0, NORM_CH, norm_step, 0)


_hop = pl.kernel(
    _hop_body,
    out_type=(
        jax.ShapeDtypeStruct((NACC, W), jnp.float32),
        jax.ShapeDtypeStruct((NACC, W), jnp.float32),
    ),
    mesh=plsc.VectorSubcoreMesh(core_axis_name="c", subcore_axis_name="s"),
    scratch_types=(
        pltpu.VMEM((BLK, CHUNK), jnp.int32),
        pltpu.VMEM((BLK, CHUNK), jnp.int32),
        pltpu.VMEM((BLK, CHUNK), jnp.int32),
        pltpu.VMEM((BLK, CHUNK), jnp.int32),
        pltpu.VMEM((CHUNK, W), jnp.float32),
        pltpu.VMEM((CHUNK, W), jnp.float32),
        pltpu.VMEM((CHUNK,), jnp.float32),
        pltpu.VMEM((ROWS_PER_TILE,), jnp.float32),
        pltpu.VMEM((CHUNK,), jnp.float32),
        pltpu.VMEM_SHARED((NACC, W), jnp.float32),
        pltpu.VMEM_SHARED((NACC, W), jnp.float32),
        pltpu.VMEM_SHARED((NACC,), jnp.float32),
        pltpu.SemaphoreType.DMA,
        pltpu.SemaphoreType.DMA,
        pltpu.SemaphoreType.DMA,
        pltpu.SemaphoreType.DMA,
        pltpu.SemaphoreType.DMA,
        pltpu.SemaphoreType.DMA,
    ),
    compiler_params=pltpu.CompilerParams(use_tc_tiling_on_sc=False),
)


def _pad_edges(src, dst):
    npad = E_PAD - E
    pad_src = (jnp.arange(npad, dtype=jnp.int32) % N)
    pad_dst = TRASH + (jnp.arange(npad, dtype=jnp.int32) % (NACC - N))
    s = jnp.concatenate([src, pad_src]).reshape(NTILES, NBLK, BLK, CHUNK)
    d = jnp.concatenate([dst, pad_dst]).reshape(NTILES, NBLK, BLK, CHUNK)
    return s, d


def kernel(x, edge_index_pa, edge_index_ap, y):
    rowpad = ((0, NACC - N), (0, 0))
    t0 = jnp.pad(x[:, : W - C], ((0, NACC - N), (0, C)))
    t1 = jnp.pad(jnp.concatenate([x[:, W - C:], y], axis=1), rowpad)

    pa_s, pa_d = _pad_edges(edge_index_pa[0], edge_index_pa[1])
    ap_s, ap_d = _pad_edges(edge_index_ap[0], edge_index_ap[1])

    h0, h1 = _hop(t0, t1, pa_s, pa_d)
    m0, m1 = _hop(h0, h1, ap_s, ap_d)

    nf = W - C
    mp = jnp.concatenate([m0[:N, :nf], m1[:N, :nf]], axis=1)
    lp = m1[:N, nf:]
    return (mp, lp)


# speedup vs baseline: 1.1378x; 1.1037x over previous
import jax
import jax.numpy as jnp
from jax import lax
from jax.experimental import pallas as pl
from jax.experimental.pallas import tpu as pltpu
from jax.experimental.pallas import tpu_sc as plsc

N = 10000
E = 320000
D = 128
C = 16

NTILES = 16
LANES = 16

ROWS_PER_TILE = 640
NACC = NTILES * ROWS_PER_TILE
TRASH = N

CHUNK = 128
NCH = 158
NCHG = NCH + 2
EPT = NCH * CHUNK
E_PAD = NTILES * EPT

W = 96
NG = W // 32

BF = jnp.bfloat16


def _hop_body(t0, t1, src3d, dst3d, out0, out1, cnt_out,
              src_idx, dst_idx, rows0, rows1, rows2, ones_b, cz,
              acc, cnt_sh,
              sem_g0, sem_g1, sem_g2, sem_s0, sem_s1, sem_s2):
    cid = lax.axis_index("c")
    sid = lax.axis_index("s")
    base_row = sid * ROWS_PER_TILE
    rslice = pl.ds(base_row, ROWS_PER_TILE)

    bufs = (rows0, rows1, rows2)
    gsems = (sem_g0, sem_g1, sem_g2)
    ssems = (sem_s0, sem_s1, sem_s2)

    zero32 = jnp.zeros((32,), BF)
    zero16 = jnp.zeros((LANES,), jnp.float32)
    one16 = jnp.ones((LANES,), jnp.float32)
    for r in range(CHUNK):
        for cc in range(NG):
            rows0[r, pl.ds(cc * 32, 32)] = zero32
    for i in range(CHUNK // LANES):
        ones_b[pl.ds(i * LANES, LANES)] = one16
    for i in range(ROWS_PER_TILE // LANES):
        cz[pl.ds(i * LANES, LANES)] = zero16

    pltpu.sync_copy(cz, cnt_sh.at[rslice])
    for k in range(ROWS_PER_TILE // CHUNK):
        pltpu.sync_copy(rows0, acc.at[pl.ds(base_row + k * CHUNK, CHUNK)])

    pltpu.sync_copy(src3d.at[sid], src_idx)
    pltpu.sync_copy(dst3d.at[sid], dst_idx)

    def gather(j, b):
        @pl.when(cid == 0)
        def _():
            pltpu.async_copy(t0.at[src_idx.at[j]], bufs[b], gsems[b])

        @pl.when(cid == 1)
        def _():
            pltpu.async_copy(t1.at[src_idx.at[j]], bufs[b], gsems[b])

    def gather_wait(b):
        pltpu.make_async_copy(
            t0.at[src_idx.at[0]], bufs[b], gsems[b]).wait()

    def scatter(j, b):
        dij = dst_idx.at[j]
        pltpu.async_copy(bufs[b], acc.at[dij], ssems[b], add=True)
        pltpu.async_copy(ones_b, cnt_sh.at[dij], ssems[b], add=True)

    def scatter_wait(b):
        pltpu.make_async_copy(bufs[b], acc.at[dst_idx.at[0]], ssems[b]).wait()
        pltpu.make_async_copy(ones_b, cnt_sh.at[dst_idx.at[0]], ssems[b]).wait()

    gather(0, 0)
    gather(1, 1)

    plsc.subcore_barrier()

    gather_wait(0)
    scatter(0, 0)
    gather(2, 2)
    gather_wait(1)
    scatter(1, 1)
    scatter_wait(0)
    gather(3, 0)

    def acc_step(i, carry):
        for b in range(3):
            j = 3 * i + 2 + b
            bj = (2 + b) % 3
            gather_wait(bj)
            scatter(j, bj)
            scatter_wait((bj + 2) % 3)
            gather(j + 2, (bj + 2) % 3)
        return carry

    lax.fori_loop(0, (NCH - 2) // 3, acc_step, 0)

    scatter_wait((NCH - 1) % 3)
    gather_wait(NCH % 3)
    gather_wait((NCH + 1) % 3)

    plsc.subcore_barrier()

    @pl.when(cid == 0)
    def _():
        pltpu.sync_copy(acc.at[rslice], out0.at[rslice])
        pltpu.sync_copy(cnt_sh.at[rslice], cnt_out.at[rslice])

    @pl.when(cid == 1)
    def _():
        pltpu.sync_copy(acc.at[rslice], out1.at[rslice])


_hop = pl.kernel(
    _hop_body,
    out_type=(
        jax.ShapeDtypeStruct((NACC, W), BF),
        jax.ShapeDtypeStruct((NACC, W), BF),
        jax.ShapeDtypeStruct((NACC,), jnp.float32),
    ),
    mesh=plsc.VectorSubcoreMesh(core_axis_name="c", subcore_axis_name="s"),
    scratch_types=(
        pltpu.VMEM((NCHG, CHUNK), jnp.int32),
        pltpu.VMEM((NCHG, CHUNK), jnp.int32),
        pltpu.VMEM((CHUNK, W), BF),
        pltpu.VMEM((CHUNK, W), BF),
        pltpu.VMEM((CHUNK, W), BF),
        pltpu.VMEM((CHUNK,), jnp.float32),
        pltpu.VMEM((ROWS_PER_TILE,), jnp.float32),
        pltpu.VMEM_SHARED((NACC, W), BF),
        pltpu.VMEM_SHARED((NACC,), jnp.float32),
        pltpu.SemaphoreType.DMA,
        pltpu.SemaphoreType.DMA,
        pltpu.SemaphoreType.DMA,
        pltpu.SemaphoreType.DMA,
        pltpu.SemaphoreType.DMA,
        pltpu.SemaphoreType.DMA,
    ),
    compiler_params=pltpu.CompilerParams(use_tc_tiling_on_sc=False),
)


def _norm_body(s0, s1, cnt, o0, o1):
    rec = 1.0 / jnp.maximum(cnt[...], 1.0)
    o0[...] = (s0[...].astype(jnp.float32) * rec).astype(o0.dtype)
    o1[...] = (s1[...].astype(jnp.float32) * rec).astype(o1.dtype)


def _make_norm(out_dt):
    return pl.pallas_call(
        _norm_body,
        out_shape=(
            jax.ShapeDtypeStruct((NACC, W), out_dt),
            jax.ShapeDtypeStruct((NACC, W), out_dt),
        ),
    )


_norm_mid = _make_norm(BF)
_norm_fin = _make_norm(jnp.float32)


def _pad_edges(src, dst):
    npad = E_PAD - E
    pad_src = (jnp.arange(npad, dtype=jnp.int32) % N)
    pad_dst = TRASH + (jnp.arange(npad, dtype=jnp.int32) % (NACC - N))
    s = jnp.concatenate([src, pad_src]).reshape(NTILES, NCH, CHUNK)
    d = jnp.concatenate([dst, pad_dst]).reshape(NTILES, NCH, CHUNK)
    dummy = jnp.zeros((NTILES, 2, CHUNK), jnp.int32)
    return (jnp.concatenate([s, dummy], axis=1),
            jnp.concatenate([d, dummy], axis=1))


def kernel(x, edge_index_pa, edge_index_ap, y):
    xb = x.astype(BF)
    nf = 64
    t0 = jnp.pad(xb[:, :nf], ((0, NACC - N), (0, W - nf)))
    t1 = jnp.pad(jnp.concatenate([xb[:, nf:], y.astype(BF)], axis=1),
                 ((0, NACC - N), (0, W - nf - C)))

    pa_s, pa_d = _pad_edges(edge_index_pa[0], edge_index_pa[1])
    ap_s, ap_d = _pad_edges(edge_index_ap[0], edge_index_ap[1])

    s0, s1, c1 = _hop(t0, t1, pa_s, pa_d)
    h0, h1 = _norm_mid(s0, s1, c1.reshape(NACC, 1))
    s2, s3, c2 = _hop(h0, h1, ap_s, ap_d)
    m0, m1 = _norm_fin(s2, s3, c2.reshape(NACC, 1))

    mp = jnp.concatenate([m0[:N, :nf], m1[:N, :nf]], axis=1)
    lp = m1[:N, nf:nf + C]
    return (mp, lp)
